# baseline (speedup 1.0000x reference)
import jax
import jax.numpy as jnp
from jax.experimental import pallas as pl

_HW = [[68, 120], [34, 60], [17, 30]]
_STRIDES = [8.0, 16.0, 32.0]
_A0 = _HW[0][0] * _HW[0][1]
_A1 = _A0 + _HW[1][0] * _HW[1][1]
_A = _A1 + _HW[2][0] * _HW[2][1]
_C = 16
_B = 64
_AP = 10752
_RPB = _AP * _C // 128
_R = _B * _RPB
_BB = 2


def _grid_xy(a_i32):
    a_i32 = jnp.minimum(a_i32, _A - 1)
    in0 = a_i32 < _A0
    in1 = a_i32 < _A1
    stride = jnp.where(in0, _STRIDES[0], jnp.where(in1, _STRIDES[1], _STRIDES[2]))
    start = jnp.where(in0, 0.0, jnp.where(in1, float(_A0), float(_A1)))
    width = jnp.where(in0, float(_HW[0][1]), jnp.where(in1, float(_HW[1][1]),
                                                       float(_HW[2][1])))
    rel = a_i32.astype(jnp.float32) - start
    gy = jnp.floor(rel / width)
    gx = rel - gy * width
    return gx, gy, stride


def _decode_kernel(x_ref, ga_ref, st_ref, o_ref, xs_ref, ys_ref, ss_ref):
    lane = jax.lax.broadcasted_iota(jnp.int32, (1, 128), 1)
    chan = lane & 15
    m2 = chan < 2
    m4 = chan < 4
    ga = ga_ref[...]
    st = st_ref[...]
    for b in range(_BB):
        x = x_ref[pl.ds(b * _RPB, _RPB), :]
        v = jnp.where(m2, x + ga, jnp.exp(x))
        o_ref[pl.ds(b * _RPB, _RPB), :] = jnp.where(m4, v * st, x)

    @pl.when(pl.program_id(0) == 0)
    def _():
        ja = jax.lax.broadcasted_iota(jnp.int32, (1, _A), 1)
        agx, agy, astride = _grid_xy(ja)
        xs_ref[...] = agx
        ys_ref[...] = agy
        ss_ref[...] = astride


def _tables():
    r = jax.lax.broadcasted_iota(jnp.int32, (_RPB, 128), 0)
    lane = jax.lax.broadcasted_iota(jnp.int32, (_RPB, 128), 1)
    a = r * 8 + (lane >> 4)
    chan = lane & 15
    gx, gy, stride = _grid_xy(a)
    ga = jnp.where(chan == 0, gx, jnp.where(chan == 1, gy, 0.0))
    return ga, stride


@jax.jit
def _decode(x):
    f32 = jnp.float32
    ga, st = _tables()
    x2 = jnp.pad(x, ((0, 0), (0, _AP - _A), (0, 0))).reshape(_R, 128)
    aux = pl.BlockSpec((_RPB, 128), lambda i: (0, 0))
    out2, xs, ys, ss = pl.pallas_call(
        _decode_kernel,
        grid=(_B // _BB,),
        in_specs=[pl.BlockSpec((_BB * _RPB, 128), lambda i: (i, 0)), aux, aux],
        out_specs=[
            pl.BlockSpec((_BB * _RPB, 128), lambda i: (i, 0)),
            pl.BlockSpec((1, _A), lambda i: (0, 0)),
            pl.BlockSpec((1, _A), lambda i: (0, 0)),
            pl.BlockSpec((1, _A), lambda i: (0, 0)),
        ],
        out_shape=[
            jax.ShapeDtypeStruct((_R, 128), f32),
            jax.ShapeDtypeStruct((1, _A), f32),
            jax.ShapeDtypeStruct((1, _A), f32),
            jax.ShapeDtypeStruct((1, _A), f32),
        ],
    )(x2, ga, st)
    out = out2.reshape(_B, _AP, _C)[:, :_A, :]
    return out, xs, ys, ss


def kernel(outputs):
    return _decode(outputs)


# speedup vs baseline: 1.7917x; 1.7917x over previous
import jax
import jax.numpy as jnp
from jax.experimental import pallas as pl

_HW = [[68, 120], [34, 60], [17, 30]]
_STRIDES = [8.0, 16.0, 32.0]
_A0 = _HW[0][0] * _HW[0][1]
_A1 = _A0 + _HW[1][0] * _HW[1][1]
_A = _A1 + _HW[2][0] * _HW[2][1]
_C = 16
_B = 64
_BBLK = 1


def _grid_xy(a_i32):
    in0 = a_i32 < _A0
    in1 = a_i32 < _A1
    stride = jnp.where(in0, _STRIDES[0], jnp.where(in1, _STRIDES[1], _STRIDES[2]))
    start = jnp.where(in0, 0.0, jnp.where(in1, float(_A0), float(_A1)))
    width = jnp.where(in0, float(_HW[0][1]), jnp.where(in1, float(_HW[1][1]),
                                                       float(_HW[2][1])))
    rel = a_i32.astype(jnp.float32) - start
    gy = jnp.floor(rel / width)
    gx = rel - gy * width
    return gx, gy, stride


def _decode_kernel(x_ref, ga_ref, st_ref, o_ref, xs_ref, ys_ref, ss_ref):
    chan = jax.lax.broadcasted_iota(jnp.int32, (1, 1, _C), 2)
    m2 = chan < 2
    m4 = chan < 4
    x = x_ref[...]
    v = jnp.where(m2, x + ga_ref[...], jnp.exp(x))
    o_ref[...] = jnp.where(m4, v * st_ref[...], x)

    @pl.when(pl.program_id(0) == 0)
    def _():
        ja = jax.lax.broadcasted_iota(jnp.int32, (1, _A), 1)
        agx, agy, astride = _grid_xy(ja)
        xs_ref[...] = agx
        ys_ref[...] = agy
        ss_ref[...] = astride


def _tables():
    a = jax.lax.broadcasted_iota(jnp.int32, (1, _A, _C), 1)
    chan = jax.lax.broadcasted_iota(jnp.int32, (1, _A, _C), 2)
    gx, gy, stride = _grid_xy(a)
    ga = jnp.where(chan == 0, gx, jnp.where(chan == 1, gy, 0.0))
    return ga, stride


@jax.jit
def _decode(x):
    f32 = jnp.float32
    ga, st = _tables()
    aux = pl.BlockSpec((1, _A, _C), lambda i: (0, 0, 0))
    out, xs, ys, ss = pl.pallas_call(
        _decode_kernel,
        grid=(_B // _BBLK,),
        in_specs=[pl.BlockSpec((_BBLK, _A, _C), lambda i: (i, 0, 0)), aux, aux],
        out_specs=[
            pl.BlockSpec((_BBLK, _A, _C), lambda i: (i, 0, 0)),
            pl.BlockSpec((1, _A), lambda i: (0, 0)),
            pl.BlockSpec((1, _A), lambda i: (0, 0)),
            pl.BlockSpec((1, _A), lambda i: (0, 0)),
        ],
        out_shape=[
            jax.ShapeDtypeStruct((_B, _A, _C), f32),
            jax.ShapeDtypeStruct((1, _A), f32),
            jax.ShapeDtypeStruct((1, _A), f32),
            jax.ShapeDtypeStruct((1, _A), f32),
        ],
    )(x, ga, st)
    return out, xs, ys, ss


def kernel(outputs):
    return _decode(outputs)
